# baseline (speedup 1.0000x reference)
import jax
import jax.numpy as jnp
from jax.experimental import pallas as pl
from jax.experimental.pallas import tpu as pltpu

_NROW = 256
_B = 16
_NCH = _NROW // _B
_D = 6


def _body(p_hbm, g_hbm, out_ref, pbuf, gbuf, sems):
    def start(k):
        slot = k % _D
        pltpu.make_async_copy(p_hbm.at[pl.ds(k * _B, _B)], pbuf.at[slot], sems.at[slot, 0]).start()
        pltpu.make_async_copy(g_hbm.at[:, pl.ds(k * _B, _B)], gbuf.at[slot], sems.at[slot, 1]).start()

    def wait(k):
        slot = k % _D
        pltpu.make_async_copy(p_hbm.at[pl.ds(k * _B, _B)], pbuf.at[slot], sems.at[slot, 0]).wait()
        pltpu.make_async_copy(g_hbm.at[:, pl.ds(k * _B, _B)], gbuf.at[slot], sems.at[slot, 1]).wait()

    for k in range(_D):
        start(k)

    s = jnp.float32(0.0)
    c = jnp.float32(0.0)
    for k in range(_NCH):
        slot = k % _D
        wait(k)
        p = pbuf[slot]
        d = gbuf[slot, 0]
        m = gbuf[slot, 1]
        sel = m > 0
        s += jnp.sum(jnp.where(sel, jnp.abs(p - d), 0.0))
        c += jnp.sum(sel.astype(jnp.float32))
        if k + _D < _NCH:
            start(k + _D)

    out_ref[0, 0] = s / c


def kernel(pred, gt, PTVs):
    del PTVs
    p3 = pred.reshape(_NROW, 128, 128)
    g4 = gt.reshape(2, _NROW, 128, 128)
    out = pl.pallas_call(
        _body,
        in_specs=[
            pl.BlockSpec(memory_space=pl.ANY),
            pl.BlockSpec(memory_space=pl.ANY),
        ],
        out_specs=pl.BlockSpec(memory_space=pltpu.SMEM),
        out_shape=jax.ShapeDtypeStruct((1, 1), jnp.float32),
        scratch_shapes=[
            pltpu.VMEM((_D, _B, 128, 128), jnp.float32),
            pltpu.VMEM((_D, 2, _B, 128, 128), jnp.float32),
            pltpu.SemaphoreType.DMA((_D, 2)),
        ],
    )(p3, g4)
    return out.reshape(())
